# baseline (speedup 1.0000x reference)
import functools

import jax
import jax.numpy as jnp
from jax import lax
from jax.experimental import pallas as pl
from jax.experimental.pallas import tpu as pltpu
from jax.experimental.pallas import tpu_sc as plsc

N = 10000
NP = 10240
E = 320000
D = 128
NORD = 10
NC, NS, L = 2, 16, 16
NW = NC * NS

ROWS_PER_TILE = NP // NS
K = 80
EPT_A = E // NS
NCH_A = EPT_A // K
EPW_C = E // NW
NCH_C = EPW_C // K


def _zero_rows(zbuf, nrows, ncols):
    zeros16 = jnp.zeros((L,), jnp.float32)

    def body(i, _):
        for j in range(ncols // L):
            zbuf[i, pl.ds(j * L, L)] = zeros16
        return 0

    lax.fori_loop(0, nrows, body, 0)


def _degrees(src1, dst1):
    mesh = plsc.VectorSubcoreMesh(core_axis_name="c", subcore_axis_name="s")

    @functools.partial(
        pl.kernel,
        mesh=mesh,
        out_type=(
            jax.ShapeDtypeStruct((NP, L), jnp.float32),
            jax.ShapeDtypeStruct((NP, L), jnp.float32),
        ),
        scratch_types=[
            pltpu.VMEM_SHARED((NP, L), jnp.float32),
            pltpu.VMEM((ROWS_PER_TILE, L), jnp.float32),
            pltpu.VMEM((K, L), jnp.float32),
            pltpu.VMEM((EPT_A,), jnp.int32),
            pltpu.SemaphoreType.DMA,
        ],
        compiler_params=pltpu.CompilerParams(use_tc_tiling_on_sc=False),
    )
    def k(src_hbm, dst_hbm, ocnt_hbm, icnt_hbm, acc, zbuf, ones_b, idx_all,
          sem):
        c = lax.axis_index("c")
        s = lax.axis_index("s")

        @pl.when(c == 0)
        def _():
            pltpu.sync_copy(src_hbm.at[pl.ds(s * EPT_A, EPT_A)], idx_all)

        @pl.when(c == 1)
        def _():
            pltpu.sync_copy(dst_hbm.at[pl.ds(s * EPT_A, EPT_A)], idx_all)

        _zero_rows(zbuf, ROWS_PER_TILE, L)
        pltpu.sync_copy(zbuf, acc.at[pl.ds(s * ROWS_PER_TILE, ROWS_PER_TILE)])
        ones16 = jnp.ones((L,), jnp.float32)
        for i in range(K):
            ones_b[i, :] = ones16
        plsc.subcore_barrier()

        def gbody(g, _):
            for u in range(5):
                pltpu.async_copy(
                    ones_b, acc.at[idx_all.at[pl.ds((g * 5 + u) * K, K)]],
                    sem, add=True)
            for u in range(5):
                pltpu.make_async_copy(ones_b, acc.at[idx_all.at[pl.ds(0, K)]],
                                      sem).wait()
            return 0

        lax.fori_loop(0, NCH_A // 5, gbody, 0)
        plsc.subcore_barrier()
        row0 = s * ROWS_PER_TILE

        @pl.when(c == 0)
        def _():
            pltpu.sync_copy(acc.at[pl.ds(row0, ROWS_PER_TILE)],
                            ocnt_hbm.at[pl.ds(row0, ROWS_PER_TILE)])

        @pl.when(c == 1)
        def _():
            pltpu.sync_copy(acc.at[pl.ds(row0, ROWS_PER_TILE)],
                            icnt_hbm.at[pl.ds(row0, ROWS_PER_TILE)])

    return k(src1, dst1)


def _gidx_body(src_ref, ord_ref, out_ref):
    out_ref[...] = ord_ref[...] * N + src_ref[...]


def _gidx(src2, ord2):
    return pl.pallas_call(
        _gidx_body,
        out_shape=jax.ShapeDtypeStruct((E // D, D), jnp.int32),
    )(src2, ord2)


def _transform_body(feat_ref, cnt_ref, w_ref, out_ref):
    cnt = cnt_ref[:, 0:1]
    scale = 1.0 / jnp.sqrt(jnp.maximum(cnt, 1.0))
    out_ref[...] = lax.dot_general(
        feat_ref[...] * scale, w_ref[0],
        dimension_numbers=(((1,), (1,)), ((), ())),
        preferred_element_type=jnp.float32)


def _transform(feat, ocnt, w3):
    blk = 1000
    nb = N // blk
    return pl.pallas_call(
        _transform_body,
        grid=(nb, NORD),
        in_specs=[
            pl.BlockSpec((blk, D), lambda i, o: (i, 0)),
            pl.BlockSpec((blk, L), lambda i, o: (i, 0)),
            pl.BlockSpec((1, D, D), lambda i, o: (o, 0, 0)),
        ],
        out_specs=pl.BlockSpec((blk, D), lambda i, o: (o * nb + i, 0)),
        out_shape=jax.ShapeDtypeStruct((NORD * N, D), jnp.float32),
    )(feat, ocnt, w3)


def _gather_scatter(tt, gidx1, dst1):
    mesh = plsc.VectorSubcoreMesh(core_axis_name="c", subcore_axis_name="s")
    SEC, CPS = 5, NCH_C // 5
    EPS = CPS * K

    @functools.partial(
        pl.kernel,
        mesh=mesh,
        out_type=jax.ShapeDtypeStruct((NC, NP, D), jnp.float32),
        scratch_types=[
            pltpu.VMEM_SHARED((NP, D), jnp.float32),
            pltpu.VMEM((EPS,), jnp.int32),
            pltpu.VMEM((EPS,), jnp.int32),
            pltpu.VMEM((K, D), jnp.float32),
            pltpu.VMEM((K, D), jnp.float32),
            pltpu.SemaphoreType.DMA,
            pltpu.SemaphoreType.DMA,
            pltpu.SemaphoreType.DMA,
            pltpu.SemaphoreType.DMA,
        ],
        compiler_params=pltpu.CompilerParams(use_tc_tiling_on_sc=False),
    )
    def k(tt_hbm, gidx_hbm, dst_hbm, part_hbm,
          acc, gidx_sec, dst_sec, rows0, rows1, gs0, gs1, ss0, ss1):
        c = lax.axis_index("c")
        s = lax.axis_index("s")
        wid = s * NC + c
        row0 = s * ROWS_PER_TILE
        ebase = wid * EPW_C

        _zero_rows(rows0, K, D)
        for r in range(8):
            pltpu.sync_copy(rows0, acc.at[pl.ds(row0 + r * K, K)])
        plsc.subcore_barrier()

        def sbody(sec, _):
            pltpu.sync_copy(gidx_hbm.at[pl.ds(ebase + sec * EPS, EPS)],
                            gidx_sec)
            pltpu.sync_copy(dst_hbm.at[pl.ds(ebase + sec * EPS, EPS)],
                            dst_sec)
            pltpu.async_copy(tt_hbm.at[gidx_sec.at[pl.ds(0, K)]], rows0, gs0)

            def body(i, _):
                def step(rcur, gcur, scur, rnxt, gnxt, snxt):
                    pltpu.make_async_copy(
                        tt_hbm.at[gidx_sec.at[pl.ds(i * K, K)]], rcur,
                        gcur).wait()

                    @pl.when(i >= 1)
                    def _():
                        pltpu.make_async_copy(
                            rnxt, acc.at[dst_sec.at[pl.ds(0, K)]],
                            snxt).wait()

                    @pl.when(i + 1 < CPS)
                    def _():
                        pltpu.async_copy(
                            tt_hbm.at[gidx_sec.at[pl.ds((i + 1) * K, K)]],
                            rnxt, gnxt)

                    pltpu.async_copy(
                        rcur, acc.at[dst_sec.at[pl.ds(i * K, K)]], scur,
                        add=True)

                @pl.when(i % 2 == 0)
                def _():
                    step(rows0, gs0, ss0, rows1, gs1, ss1)

                @pl.when(i % 2 == 1)
                def _():
                    step(rows1, gs1, ss1, rows0, gs0, ss0)

                return 0

            lax.fori_loop(0, CPS, body, 0)
            pltpu.make_async_copy(rows0, acc.at[dst_sec.at[pl.ds(0, K)]],
                                  ss0).wait()
            return 0

        lax.fori_loop(0, SEC, sbody, 0)
        plsc.subcore_barrier()

        @pl.when(c == 0)
        def _():
            pltpu.sync_copy(acc.at[pl.ds(row0, ROWS_PER_TILE)],
                            part_hbm.at[0, pl.ds(row0, ROWS_PER_TILE)])

        @pl.when(c == 1)
        def _():
            pltpu.sync_copy(acc.at[pl.ds(row0, ROWS_PER_TILE)],
                            part_hbm.at[1, pl.ds(row0, ROWS_PER_TILE)])

    return k(tt, gidx1, dst1)


def _final_body(part_ref, cnt_ref, bias_ref, out_ref):
    p = part_ref[0] + part_ref[1]
    cnt = cnt_ref[:, 0:1]
    scale = 1.0 / jnp.sqrt(jnp.maximum(cnt, 1.0))
    out_ref[...] = p * scale + bias_ref[...]


def _finalize(part, icnt, bias2d):
    blk = 1000
    return pl.pallas_call(
        _final_body,
        grid=(N // blk,),
        in_specs=[
            pl.BlockSpec((NC, blk, D), lambda i: (0, i, 0)),
            pl.BlockSpec((blk, L), lambda i: (i, 0)),
            pl.BlockSpec((1, D), lambda i: (0, 0)),
        ],
        out_specs=pl.BlockSpec((blk, D), lambda i: (i, 0)),
        out_shape=jax.ShapeDtypeStruct((N, D), jnp.float32),
    )(part, icnt, bias2d)


def kernel(feat, edge_index, edge_order, o_emb_weight, bias):
    ei = edge_index.astype(jnp.int32)
    src = ei[0]
    dst = ei[1]
    order = edge_order.astype(jnp.int32)
    ocnt, icnt = _degrees(src, dst)
    gidx = _gidx(src.reshape(E // D, D), order.reshape(E // D, D))
    t = _transform(feat, ocnt, o_emb_weight.reshape(NORD, D, D))
    part = _gather_scatter(t, gidx.reshape(E), dst)
    return _finalize(part, icnt, bias.reshape(1, D))


# speedup vs baseline: 1.2535x; 1.2535x over previous
import functools

import jax
import jax.numpy as jnp
from jax import lax
from jax.experimental import pallas as pl
from jax.experimental.pallas import tpu as pltpu
from jax.experimental.pallas import tpu_sc as plsc

N = 10000
NP = 10240
E = 320000
D = 128
NORD = 10
NC, NS, L = 2, 16, 16
NW = NC * NS

ROWS_PER_TILE = NP // NS
K = 80
EPT_A = E // NS
NCH_A = EPT_A // K
EPW_C = E // NW
NCH_C = EPW_C // K


def _zero_rows(zbuf, nrows, ncols):
    zeros16 = jnp.zeros((L,), jnp.float32)

    def body(i, _):
        for j in range(ncols // L):
            zbuf[i, pl.ds(j * L, L)] = zeros16
        return 0

    lax.fori_loop(0, nrows, body, 0)


def _degrees(src1, dst1):
    mesh = plsc.VectorSubcoreMesh(core_axis_name="c", subcore_axis_name="s")

    @functools.partial(
        pl.kernel,
        mesh=mesh,
        out_type=(
            jax.ShapeDtypeStruct((NP, L), jnp.float32),
            jax.ShapeDtypeStruct((NP, L), jnp.float32),
        ),
        scratch_types=[
            pltpu.VMEM_SHARED((NP, L), jnp.float32),
            pltpu.VMEM((ROWS_PER_TILE, L), jnp.float32),
            pltpu.VMEM((K, L), jnp.float32),
            pltpu.VMEM((EPT_A,), jnp.int32),
            pltpu.SemaphoreType.DMA,
        ],
        compiler_params=pltpu.CompilerParams(use_tc_tiling_on_sc=False),
    )
    def k(src_hbm, dst_hbm, ocnt_hbm, icnt_hbm, acc, zbuf, ones_b, idx_all,
          sem):
        c = lax.axis_index("c")
        s = lax.axis_index("s")

        @pl.when(c == 0)
        def _():
            pltpu.sync_copy(src_hbm.at[pl.ds(s * EPT_A, EPT_A)], idx_all)

        @pl.when(c == 1)
        def _():
            pltpu.sync_copy(dst_hbm.at[pl.ds(s * EPT_A, EPT_A)], idx_all)

        _zero_rows(zbuf, ROWS_PER_TILE, L)
        pltpu.sync_copy(zbuf, acc.at[pl.ds(s * ROWS_PER_TILE, ROWS_PER_TILE)])
        ones16 = jnp.ones((L,), jnp.float32)
        for i in range(K):
            ones_b[i, :] = ones16
        plsc.subcore_barrier()

        def gbody(g, _):
            for u in range(5):
                pltpu.async_copy(
                    ones_b, acc.at[idx_all.at[pl.ds((g * 5 + u) * K, K)]],
                    sem, add=True)
            for u in range(5):
                pltpu.make_async_copy(ones_b, acc.at[idx_all.at[pl.ds(0, K)]],
                                      sem).wait()
            return 0

        lax.fori_loop(0, NCH_A // 5, gbody, 0)
        plsc.subcore_barrier()
        row0 = s * ROWS_PER_TILE

        @pl.when(c == 0)
        def _():
            pltpu.sync_copy(acc.at[pl.ds(row0, ROWS_PER_TILE)],
                            ocnt_hbm.at[pl.ds(row0, ROWS_PER_TILE)])

        @pl.when(c == 1)
        def _():
            pltpu.sync_copy(acc.at[pl.ds(row0, ROWS_PER_TILE)],
                            icnt_hbm.at[pl.ds(row0, ROWS_PER_TILE)])

    return k(src1, dst1)


def _gidx_body(ei_ref, ord_ref, gidx_ref, src_ref, dst_ref):
    s = ei_ref[0]
    gidx_ref[...] = ord_ref[...] * N + s
    src_ref[...] = s
    dst_ref[...] = ei_ref[1]


def _gidx(ei3, ord2):
    R = E // D
    return pl.pallas_call(
        _gidx_body,
        out_shape=(
            jax.ShapeDtypeStruct((R, D), jnp.int32),
            jax.ShapeDtypeStruct((R, D), jnp.int32),
            jax.ShapeDtypeStruct((R, D), jnp.int32),
        ),
    )(ei3, ord2)


def _transform_body(feat_ref, cnt_ref, wt_ref, out_ref):
    cnt = cnt_ref[:, 0:1]
    scale = 1.0 / jnp.sqrt(jnp.maximum(cnt, 1.0))
    res = jnp.dot(feat_ref[...] * scale, wt_ref[...],
                  preferred_element_type=jnp.float32)
    for o in range(NORD):
        out_ref[o] = res[:, o * D:(o + 1) * D]


def _transform(feat, ocnt, wt):
    blk = 1000
    return pl.pallas_call(
        _transform_body,
        grid=(N // blk,),
        in_specs=[
            pl.BlockSpec((blk, D), lambda i: (i, 0)),
            pl.BlockSpec((blk, L), lambda i: (i, 0)),
            pl.BlockSpec((D, NORD * D), lambda i: (0, 0)),
        ],
        out_specs=pl.BlockSpec((NORD, blk, D), lambda i: (0, i, 0)),
        out_shape=jax.ShapeDtypeStruct((NORD, N, D), jnp.float32),
    )(feat, ocnt, wt)


def _gather_scatter(tt, gidx1, dst1):
    mesh = plsc.VectorSubcoreMesh(core_axis_name="c", subcore_axis_name="s")
    SEC, CPS = 5, NCH_C // 5
    EPS = CPS * K

    @functools.partial(
        pl.kernel,
        mesh=mesh,
        out_type=jax.ShapeDtypeStruct((NC, NP, D), jnp.float32),
        scratch_types=[
            pltpu.VMEM_SHARED((NP, D), jnp.float32),
            pltpu.VMEM((EPS,), jnp.int32),
            pltpu.VMEM((EPS,), jnp.int32),
            pltpu.VMEM((K, D), jnp.float32),
            pltpu.VMEM((K, D), jnp.float32),
            pltpu.SemaphoreType.DMA,
            pltpu.SemaphoreType.DMA,
            pltpu.SemaphoreType.DMA,
            pltpu.SemaphoreType.DMA,
        ],
        compiler_params=pltpu.CompilerParams(use_tc_tiling_on_sc=False),
    )
    def k(tt_hbm, gidx_hbm, dst_hbm, part_hbm,
          acc, gidx_sec, dst_sec, rows0, rows1, gs0, gs1, ss0, ss1):
        c = lax.axis_index("c")
        s = lax.axis_index("s")
        wid = s * NC + c
        row0 = s * ROWS_PER_TILE
        ebase = wid * EPW_C

        _zero_rows(rows0, K, D)
        for r in range(8):
            pltpu.sync_copy(rows0, acc.at[pl.ds(row0 + r * K, K)])
        plsc.subcore_barrier()

        def sbody(sec, _):
            pltpu.sync_copy(gidx_hbm.at[pl.ds(ebase + sec * EPS, EPS)],
                            gidx_sec)
            pltpu.sync_copy(dst_hbm.at[pl.ds(ebase + sec * EPS, EPS)],
                            dst_sec)
            pltpu.async_copy(tt_hbm.at[gidx_sec.at[pl.ds(0, K)]], rows0, gs0)

            def body(i, _):
                def step(rcur, gcur, scur, rnxt, gnxt, snxt):
                    pltpu.make_async_copy(
                        tt_hbm.at[gidx_sec.at[pl.ds(i * K, K)]], rcur,
                        gcur).wait()

                    @pl.when(i >= 1)
                    def _():
                        pltpu.make_async_copy(
                            rnxt, acc.at[dst_sec.at[pl.ds(0, K)]],
                            snxt).wait()

                    @pl.when(i + 1 < CPS)
                    def _():
                        pltpu.async_copy(
                            tt_hbm.at[gidx_sec.at[pl.ds((i + 1) * K, K)]],
                            rnxt, gnxt)

                    pltpu.async_copy(
                        rcur, acc.at[dst_sec.at[pl.ds(i * K, K)]], scur,
                        add=True)

                @pl.when(i % 2 == 0)
                def _():
                    step(rows0, gs0, ss0, rows1, gs1, ss1)

                @pl.when(i % 2 == 1)
                def _():
                    step(rows1, gs1, ss1, rows0, gs0, ss0)

                return 0

            lax.fori_loop(0, CPS, body, 0)
            pltpu.make_async_copy(rows0, acc.at[dst_sec.at[pl.ds(0, K)]],
                                  ss0).wait()
            return 0

        lax.fori_loop(0, SEC, sbody, 0)
        plsc.subcore_barrier()

        @pl.when(c == 0)
        def _():
            pltpu.sync_copy(acc.at[pl.ds(row0, ROWS_PER_TILE)],
                            part_hbm.at[0, pl.ds(row0, ROWS_PER_TILE)])

        @pl.when(c == 1)
        def _():
            pltpu.sync_copy(acc.at[pl.ds(row0, ROWS_PER_TILE)],
                            part_hbm.at[1, pl.ds(row0, ROWS_PER_TILE)])

    return k(tt, gidx1, dst1)


def _final_body(part_ref, cnt_ref, bias_ref, out_ref):
    p = part_ref[0] + part_ref[1]
    cnt = cnt_ref[:, 0:1]
    scale = 1.0 / jnp.sqrt(jnp.maximum(cnt, 1.0))
    out_ref[...] = p * scale + bias_ref[...]


def _finalize(part, icnt, bias2d):
    blk = 1000
    return pl.pallas_call(
        _final_body,
        grid=(N // blk,),
        in_specs=[
            pl.BlockSpec((NC, blk, D), lambda i: (0, i, 0)),
            pl.BlockSpec((blk, L), lambda i: (i, 0)),
            pl.BlockSpec((1, D), lambda i: (0, 0)),
        ],
        out_specs=pl.BlockSpec((blk, D), lambda i: (i, 0)),
        out_shape=jax.ShapeDtypeStruct((N, D), jnp.float32),
    )(part, icnt, bias2d)


def kernel(feat, edge_index, edge_order, o_emb_weight, bias):
    ei3 = edge_index.astype(jnp.int32).reshape(2, E // D, D)
    order = edge_order.astype(jnp.int32)
    gidx2, src2, dst2 = _gidx(ei3, order.reshape(E // D, D))
    src1 = src2.reshape(E)
    dst1 = dst2.reshape(E)
    ocnt, icnt = _degrees(src1, dst1)
    wt = o_emb_weight.reshape(NORD, D, D).transpose(2, 0, 1).reshape(D, NORD * D)
    t = _transform(feat, ocnt, wt)
    part = _gather_scatter(t.reshape(NORD * N, D), gidx2.reshape(E), dst1)
    return _finalize(part, icnt, bias.reshape(1, D))


# speedup vs baseline: 1.6117x; 1.2858x over previous
import functools

import jax
import jax.numpy as jnp
from jax import lax
from jax.experimental import pallas as pl
from jax.experimental.pallas import tpu as pltpu
from jax.experimental.pallas import tpu_sc as plsc

N = 10000
NP = 10240
E = 320000
D = 128
NORD = 10
NC, NS, L = 2, 16, 16
NW = NC * NS

ROWS_PER_TILE = NP // NS
K = 80
EPT_A = E // NS
NCH_A = EPT_A // K
EPW_C = E // NW
NCH_C = EPW_C // K


def _zero_rows(zbuf, nrows, ncols):
    zeros16 = jnp.zeros((L,), jnp.float32)

    def body(i, _):
        for j in range(ncols // L):
            zbuf[i, pl.ds(j * L, L)] = zeros16
        return 0

    lax.fori_loop(0, nrows, body, 0)


def _degrees(src1, dst1):
    mesh = plsc.VectorSubcoreMesh(core_axis_name="c", subcore_axis_name="s")

    @functools.partial(
        pl.kernel,
        mesh=mesh,
        out_type=(
            jax.ShapeDtypeStruct((NP, L), jnp.float32),
            jax.ShapeDtypeStruct((NP, L), jnp.float32),
        ),
        scratch_types=[
            pltpu.VMEM_SHARED((NP, L), jnp.float32),
            pltpu.VMEM((ROWS_PER_TILE, L), jnp.float32),
            pltpu.VMEM((K, L), jnp.float32),
            pltpu.VMEM((EPT_A,), jnp.int32),
            pltpu.SemaphoreType.DMA,
        ],
        compiler_params=pltpu.CompilerParams(use_tc_tiling_on_sc=False),
    )
    def k(src_hbm, dst_hbm, ocnt_hbm, icnt_hbm, acc, zbuf, ones_b, idx_all,
          sem):
        c = lax.axis_index("c")
        s = lax.axis_index("s")

        @pl.when(c == 0)
        def _():
            pltpu.sync_copy(src_hbm.at[pl.ds(s * EPT_A, EPT_A)], idx_all)

        @pl.when(c == 1)
        def _():
            pltpu.sync_copy(dst_hbm.at[pl.ds(s * EPT_A, EPT_A)], idx_all)

        _zero_rows(zbuf, ROWS_PER_TILE, L)
        pltpu.sync_copy(zbuf, acc.at[pl.ds(s * ROWS_PER_TILE, ROWS_PER_TILE)])
        ones16 = jnp.ones((L,), jnp.float32)
        for i in range(K):
            ones_b[i, :] = ones16
        plsc.subcore_barrier()

        def gbody(g, _):
            for u in range(5):
                pltpu.async_copy(
                    ones_b, acc.at[idx_all.at[pl.ds((g * 5 + u) * K, K)]],
                    sem, add=True)
            for u in range(5):
                pltpu.make_async_copy(ones_b, acc.at[idx_all.at[pl.ds(0, K)]],
                                      sem).wait()
            return 0

        lax.fori_loop(0, NCH_A // 5, gbody, 0)
        plsc.subcore_barrier()
        row0 = s * ROWS_PER_TILE

        @pl.when(c == 0)
        def _():
            pltpu.sync_copy(acc.at[pl.ds(row0, ROWS_PER_TILE)],
                            ocnt_hbm.at[pl.ds(row0, ROWS_PER_TILE)])

        @pl.when(c == 1)
        def _():
            pltpu.sync_copy(acc.at[pl.ds(row0, ROWS_PER_TILE)],
                            icnt_hbm.at[pl.ds(row0, ROWS_PER_TILE)])

    return k(src1, dst1)


def _gidx_body(ei_ref, ord_ref, gidx_ref, src_ref, dst_ref):
    s = ei_ref[0]
    gidx_ref[...] = ord_ref[...] * N + s
    src_ref[...] = s
    dst_ref[...] = ei_ref[1]


def _gidx(ei3, ord2):
    R = E // D
    return pl.pallas_call(
        _gidx_body,
        out_shape=(
            jax.ShapeDtypeStruct((R, D), jnp.int32),
            jax.ShapeDtypeStruct((R, D), jnp.int32),
            jax.ShapeDtypeStruct((R, D), jnp.int32),
        ),
    )(ei3, ord2)


def _transform_body(feat_ref, cnt_ref, wt_ref, out_ref):
    cnt = cnt_ref[:, 0:1]
    scale = 1.0 / jnp.sqrt(jnp.maximum(cnt, 1.0))
    res = jnp.dot(feat_ref[...] * scale, wt_ref[...],
                  preferred_element_type=jnp.float32)
    for o in range(NORD):
        out_ref[o] = res[:, o * D:(o + 1) * D]


def _transform(feat, ocnt, wt):
    blk = 1000
    return pl.pallas_call(
        _transform_body,
        grid=(N // blk,),
        in_specs=[
            pl.BlockSpec((blk, D), lambda i: (i, 0)),
            pl.BlockSpec((blk, L), lambda i: (i, 0)),
            pl.BlockSpec((D, NORD * D), lambda i: (0, 0)),
        ],
        out_specs=pl.BlockSpec((NORD, blk, D), lambda i: (0, i, 0)),
        out_shape=jax.ShapeDtypeStruct((NORD, N, D), jnp.float32),
    )(feat, ocnt, wt)


def _gather_scatter(tt, gidx1, dst1):
    mesh = plsc.VectorSubcoreMesh(core_axis_name="c", subcore_axis_name="s")
    SEC, CPS = 5, NCH_C // 5
    EPS = CPS * K

    @functools.partial(
        pl.kernel,
        mesh=mesh,
        out_type=jax.ShapeDtypeStruct((NC, NP, D), jnp.float32),
        scratch_types=[
            pltpu.VMEM_SHARED((NP, D), jnp.float32),
            pltpu.VMEM((EPS,), jnp.int32),
            pltpu.VMEM((EPS,), jnp.int32),
            pltpu.VMEM((K, D), jnp.float32),
            pltpu.VMEM((K, D), jnp.float32),
            pltpu.VMEM((K, D), jnp.float32),
            pltpu.SemaphoreType.DMA,
            pltpu.SemaphoreType.DMA,
            pltpu.SemaphoreType.DMA,
        ],
        compiler_params=pltpu.CompilerParams(use_tc_tiling_on_sc=False),
    )
    def k(tt_hbm, gidx_hbm, dst_hbm, part_hbm,
          acc, gidx_sec, dst_sec, rows0, rows1, rows2, gs0, gs1, gs2):
        c = lax.axis_index("c")
        s = lax.axis_index("s")
        wid = s * NC + c
        row0 = s * ROWS_PER_TILE
        ebase = wid * EPW_C

        _zero_rows(rows0, K, D)
        for r in range(8):
            pltpu.sync_copy(rows0, acc.at[pl.ds(row0 + r * K, K)])
        plsc.subcore_barrier()

        def sbody(sec, _):
            pltpu.sync_copy(gidx_hbm.at[pl.ds(ebase + sec * EPS, EPS)],
                            gidx_sec)
            pltpu.sync_copy(dst_hbm.at[pl.ds(ebase + sec * EPS, EPS)],
                            dst_sec)
            pltpu.async_copy(tt_hbm.at[gidx_sec.at[pl.ds(0, K)]], rows0, gs0)
            pltpu.async_copy(tt_hbm.at[gidx_sec.at[pl.ds(K, K)]], rows1, gs1)

            def body(i, _):
                def step(rcur, gcur, rfar, gfar):
                    pltpu.make_async_copy(
                        tt_hbm.at[gidx_sec.at[pl.ds(i * K, K)]], rcur,
                        gcur).wait()

                    @pl.when(i + 2 < CPS)
                    def _():
                        pltpu.async_copy(
                            tt_hbm.at[gidx_sec.at[pl.ds((i + 2) * K, K)]],
                            rfar, gfar)

                    pltpu.sync_copy(rcur, acc.at[dst_sec.at[pl.ds(i * K, K)]],
                                    add=True)

                @pl.when(i % 3 == 0)
                def _():
                    step(rows0, gs0, rows2, gs2)

                @pl.when(i % 3 == 1)
                def _():
                    step(rows1, gs1, rows0, gs0)

                @pl.when(i % 3 == 2)
                def _():
                    step(rows2, gs2, rows1, gs1)

                return 0

            lax.fori_loop(0, CPS, body, 0)
            return 0

        lax.fori_loop(0, SEC, sbody, 0)
        plsc.subcore_barrier()

        @pl.when(c == 0)
        def _():
            pltpu.sync_copy(acc.at[pl.ds(row0, ROWS_PER_TILE)],
                            part_hbm.at[0, pl.ds(row0, ROWS_PER_TILE)])

        @pl.when(c == 1)
        def _():
            pltpu.sync_copy(acc.at[pl.ds(row0, ROWS_PER_TILE)],
                            part_hbm.at[1, pl.ds(row0, ROWS_PER_TILE)])

    return k(tt, gidx1, dst1)


def _final_body(part_ref, cnt_ref, bias_ref, out_ref):
    p = part_ref[0] + part_ref[1]
    cnt = cnt_ref[:, 0:1]
    scale = 1.0 / jnp.sqrt(jnp.maximum(cnt, 1.0))
    out_ref[...] = p * scale + bias_ref[...]


def _finalize(part, icnt, bias2d):
    blk = 1000
    return pl.pallas_call(
        _final_body,
        grid=(N // blk,),
        in_specs=[
            pl.BlockSpec((NC, blk, D), lambda i: (0, i, 0)),
            pl.BlockSpec((blk, L), lambda i: (i, 0)),
            pl.BlockSpec((1, D), lambda i: (0, 0)),
        ],
        out_specs=pl.BlockSpec((blk, D), lambda i: (i, 0)),
        out_shape=jax.ShapeDtypeStruct((N, D), jnp.float32),
    )(part, icnt, bias2d)


def kernel(feat, edge_index, edge_order, o_emb_weight, bias):
    ei3 = edge_index.astype(jnp.int32).reshape(2, E // D, D)
    order = edge_order.astype(jnp.int32)
    gidx2, src2, dst2 = _gidx(ei3, order.reshape(E // D, D))
    src1 = src2.reshape(E)
    dst1 = dst2.reshape(E)
    ocnt, icnt = _degrees(src1, dst1)
    wt = o_emb_weight.reshape(NORD, D, D).transpose(2, 0, 1).reshape(D, NORD * D)
    t = _transform(feat, ocnt, wt)
    part = _gather_scatter(t.reshape(NORD * N, D), gidx2.reshape(E), dst1)
    return _finalize(part, icnt, bias.reshape(1, D))


# speedup vs baseline: 1.6468x; 1.0218x over previous
import functools

import jax
import jax.numpy as jnp
from jax import lax
from jax.experimental import pallas as pl
from jax.experimental.pallas import tpu as pltpu
from jax.experimental.pallas import tpu_sc as plsc

N = 10000
NP = 10240
E = 320000
D = 128
NORD = 10
NC, NS, L = 2, 16, 16
NW = NC * NS

ROWS_PER_TILE = NP // NS
K = 80
EPT_A = E // NS
NCH_A = EPT_A // K
EPW_C = E // NW
NCH_C = EPW_C // K


def _zero_rows(zbuf, nrows, ncols):
    zeros16 = jnp.zeros((L,), jnp.float32)

    def body(i, _):
        for j in range(ncols // L):
            zbuf[i, pl.ds(j * L, L)] = zeros16
        return 0

    lax.fori_loop(0, nrows, body, 0)


def _degrees(src1, dst1):
    mesh = plsc.VectorSubcoreMesh(core_axis_name="c", subcore_axis_name="s")

    @functools.partial(
        pl.kernel,
        mesh=mesh,
        out_type=(
            jax.ShapeDtypeStruct((NP, L), jnp.float32),
            jax.ShapeDtypeStruct((NP, L), jnp.float32),
        ),
        scratch_types=[
            pltpu.VMEM_SHARED((NP, L), jnp.float32),
            pltpu.VMEM((ROWS_PER_TILE, L), jnp.float32),
            pltpu.VMEM((K, L), jnp.float32),
            pltpu.VMEM((EPT_A,), jnp.int32),
            pltpu.SemaphoreType.DMA,
        ],
        compiler_params=pltpu.CompilerParams(use_tc_tiling_on_sc=False),
    )
    def k(src_hbm, dst_hbm, ocnt_hbm, icnt_hbm, acc, zbuf, ones_b, idx_all,
          sem):
        c = lax.axis_index("c")
        s = lax.axis_index("s")

        @pl.when(c == 0)
        def _():
            pltpu.sync_copy(src_hbm.at[pl.ds(s * EPT_A, EPT_A)], idx_all)

        @pl.when(c == 1)
        def _():
            pltpu.sync_copy(dst_hbm.at[pl.ds(s * EPT_A, EPT_A)], idx_all)

        _zero_rows(zbuf, ROWS_PER_TILE, L)
        pltpu.sync_copy(zbuf, acc.at[pl.ds(s * ROWS_PER_TILE, ROWS_PER_TILE)])
        ones16 = jnp.ones((L,), jnp.float32)
        for i in range(K):
            ones_b[i, :] = ones16
        plsc.subcore_barrier()

        def gbody(g, _):
            for u in range(5):
                pltpu.async_copy(
                    ones_b, acc.at[idx_all.at[pl.ds((g * 5 + u) * K, K)]],
                    sem, add=True)
            for u in range(5):
                pltpu.make_async_copy(ones_b, acc.at[idx_all.at[pl.ds(0, K)]],
                                      sem).wait()
            return 0

        lax.fori_loop(0, NCH_A // 5, gbody, 0)
        plsc.subcore_barrier()
        row0 = s * ROWS_PER_TILE

        @pl.when(c == 0)
        def _():
            pltpu.sync_copy(acc.at[pl.ds(row0, ROWS_PER_TILE)],
                            ocnt_hbm.at[pl.ds(row0, ROWS_PER_TILE)])

        @pl.when(c == 1)
        def _():
            pltpu.sync_copy(acc.at[pl.ds(row0, ROWS_PER_TILE)],
                            icnt_hbm.at[pl.ds(row0, ROWS_PER_TILE)])

    return k(src1, dst1)


def _gidx_body(ei_ref, ord_ref, gidx_ref, src_ref, dst_ref):
    s = ei_ref[0]
    gidx_ref[...] = ord_ref[...] * N + s
    src_ref[...] = s
    dst_ref[...] = ei_ref[1]


def _gidx(ei3, ord2):
    R = E // D
    return pl.pallas_call(
        _gidx_body,
        out_shape=(
            jax.ShapeDtypeStruct((R, D), jnp.int32),
            jax.ShapeDtypeStruct((R, D), jnp.int32),
            jax.ShapeDtypeStruct((R, D), jnp.int32),
        ),
    )(ei3, ord2)


def _transform_body(feat_ref, cnt_ref, wt_ref, out_ref):
    cnt = cnt_ref[:, 0:1]
    scale = 1.0 / jnp.sqrt(jnp.maximum(cnt, 1.0))
    res = jnp.dot(feat_ref[...] * scale, wt_ref[...],
                  preferred_element_type=jnp.float32)
    for o in range(NORD):
        out_ref[o] = res[:, o * D:(o + 1) * D]


def _transform(feat, ocnt, wt):
    blk = 1000
    return pl.pallas_call(
        _transform_body,
        grid=(N // blk,),
        in_specs=[
            pl.BlockSpec((blk, D), lambda i: (i, 0)),
            pl.BlockSpec((blk, L), lambda i: (i, 0)),
            pl.BlockSpec((D, NORD * D), lambda i: (0, 0)),
        ],
        out_specs=pl.BlockSpec((NORD, blk, D), lambda i: (0, i, 0)),
        out_shape=jax.ShapeDtypeStruct((NORD, N, D), jnp.float32),
    )(feat, ocnt, wt)


def _gather_scatter(tt, gidx1, dst1):
    mesh = plsc.VectorSubcoreMesh(core_axis_name="c", subcore_axis_name="s")
    SEC, CPS = 5, NCH_C // 5
    EPS = CPS * K

    @functools.partial(
        pl.kernel,
        mesh=mesh,
        out_type=jax.ShapeDtypeStruct((NC, NP, D), jnp.float32),
        scratch_types=[
            pltpu.VMEM_SHARED((NP, D), jnp.float32),
            pltpu.VMEM((EPS,), jnp.int32),
            pltpu.VMEM((EPS,), jnp.int32),
            pltpu.VMEM((K, D), jnp.float32),
            pltpu.VMEM((K, D), jnp.float32),
            pltpu.VMEM((K, D), jnp.float32),
            pltpu.VMEM((K, D), jnp.float32),
            pltpu.SemaphoreType.DMA,
            pltpu.SemaphoreType.DMA,
            pltpu.SemaphoreType.DMA,
            pltpu.SemaphoreType.DMA,
        ],
        compiler_params=pltpu.CompilerParams(use_tc_tiling_on_sc=False),
    )
    def k(tt_hbm, gidx_hbm, dst_hbm, part_hbm,
          acc, gidx_sec, dst_sec, rows0, rows1, rows2, rows3,
          gs0, gs1, gs2, gs3):
        c = lax.axis_index("c")
        s = lax.axis_index("s")
        wid = s * NC + c
        row0 = s * ROWS_PER_TILE
        ebase = wid * EPW_C

        _zero_rows(rows0, K, D)
        for r in range(8):
            pltpu.sync_copy(rows0, acc.at[pl.ds(row0 + r * K, K)])
        plsc.subcore_barrier()

        def sbody(sec, _):
            pltpu.sync_copy(gidx_hbm.at[pl.ds(ebase + sec * EPS, EPS)],
                            gidx_sec)
            pltpu.sync_copy(dst_hbm.at[pl.ds(ebase + sec * EPS, EPS)],
                            dst_sec)
            pltpu.async_copy(tt_hbm.at[gidx_sec.at[pl.ds(0, K)]], rows0, gs0)
            pltpu.async_copy(tt_hbm.at[gidx_sec.at[pl.ds(K, K)]], rows1, gs1)
            pltpu.async_copy(tt_hbm.at[gidx_sec.at[pl.ds(2 * K, K)]], rows2,
                             gs2)

            def body(i, _):
                def step(rcur, gcur, rfar, gfar):
                    pltpu.make_async_copy(
                        tt_hbm.at[gidx_sec.at[pl.ds(i * K, K)]], rcur,
                        gcur).wait()

                    @pl.when(i + 3 < CPS)
                    def _():
                        pltpu.async_copy(
                            tt_hbm.at[gidx_sec.at[pl.ds((i + 3) * K, K)]],
                            rfar, gfar)

                    pltpu.sync_copy(rcur, acc.at[dst_sec.at[pl.ds(i * K, K)]],
                                    add=True)

                @pl.when(i % 4 == 0)
                def _():
                    step(rows0, gs0, rows3, gs3)

                @pl.when(i % 4 == 1)
                def _():
                    step(rows1, gs1, rows0, gs0)

                @pl.when(i % 4 == 2)
                def _():
                    step(rows2, gs2, rows1, gs1)

                @pl.when(i % 4 == 3)
                def _():
                    step(rows3, gs3, rows2, gs2)

                return 0

            lax.fori_loop(0, CPS, body, 0)
            return 0

        lax.fori_loop(0, SEC, sbody, 0)
        plsc.subcore_barrier()

        @pl.when(c == 0)
        def _():
            pltpu.sync_copy(acc.at[pl.ds(row0, ROWS_PER_TILE)],
                            part_hbm.at[0, pl.ds(row0, ROWS_PER_TILE)])

        @pl.when(c == 1)
        def _():
            pltpu.sync_copy(acc.at[pl.ds(row0, ROWS_PER_TILE)],
                            part_hbm.at[1, pl.ds(row0, ROWS_PER_TILE)])

    return k(tt, gidx1, dst1)


def _final_body(part_ref, cnt_ref, bias_ref, out_ref):
    p = part_ref[0] + part_ref[1]
    cnt = cnt_ref[:, 0:1]
    scale = 1.0 / jnp.sqrt(jnp.maximum(cnt, 1.0))
    out_ref[...] = p * scale + bias_ref[...]


def _finalize(part, icnt, bias2d):
    blk = 1000
    return pl.pallas_call(
        _final_body,
        grid=(N // blk,),
        in_specs=[
            pl.BlockSpec((NC, blk, D), lambda i: (0, i, 0)),
            pl.BlockSpec((blk, L), lambda i: (i, 0)),
            pl.BlockSpec((1, D), lambda i: (0, 0)),
        ],
        out_specs=pl.BlockSpec((blk, D), lambda i: (i, 0)),
        out_shape=jax.ShapeDtypeStruct((N, D), jnp.float32),
    )(part, icnt, bias2d)


def kernel(feat, edge_index, edge_order, o_emb_weight, bias):
    ei3 = edge_index.astype(jnp.int32).reshape(2, E // D, D)
    order = edge_order.astype(jnp.int32)
    gidx2, src2, dst2 = _gidx(ei3, order.reshape(E // D, D))
    src1 = src2.reshape(E)
    dst1 = dst2.reshape(E)
    ocnt, icnt = _degrees(src1, dst1)
    wt = o_emb_weight.reshape(NORD, D, D).transpose(2, 0, 1).reshape(D, NORD * D)
    t = _transform(feat, ocnt, wt)
    part = _gather_scatter(t.reshape(NORD * N, D), gidx2.reshape(E), dst1)
    return _finalize(part, icnt, bias.reshape(1, D))


# speedup vs baseline: 1.6567x; 1.0060x over previous
import functools

import jax
import jax.numpy as jnp
from jax import lax
from jax.experimental import pallas as pl
from jax.experimental.pallas import tpu as pltpu
from jax.experimental.pallas import tpu_sc as plsc

N = 10000
NP = 10240
E = 320000
D = 128
NORD = 10
NC, NS, L = 2, 16, 16
NW = NC * NS

ROWS_PER_TILE = NP // NS
K = 80
EPT_A = E // NS
NCH_A = EPT_A // K
EPW_C = E // NW
NCH_C = EPW_C // K


def _zero_rows(zbuf, nrows, ncols):
    zeros16 = jnp.zeros((L,), jnp.float32)

    def body(i, _):
        for j in range(ncols // L):
            zbuf[i, pl.ds(j * L, L)] = zeros16
        return 0

    lax.fori_loop(0, nrows, body, 0)


def _degrees(src1, dst1):
    mesh = plsc.VectorSubcoreMesh(core_axis_name="c", subcore_axis_name="s")

    @functools.partial(
        pl.kernel,
        mesh=mesh,
        out_type=(
            jax.ShapeDtypeStruct((NP, D), jnp.float32),
            jax.ShapeDtypeStruct((NP, D), jnp.float32),
        ),
        scratch_types=[
            pltpu.VMEM_SHARED((NP, L), jnp.float32),
            pltpu.VMEM((ROWS_PER_TILE, L), jnp.float32),
            pltpu.VMEM((K, L), jnp.float32),
            pltpu.VMEM((EPT_A,), jnp.int32),
            pltpu.SemaphoreType.DMA,
        ],
        compiler_params=pltpu.CompilerParams(use_tc_tiling_on_sc=False),
    )
    def k(src_hbm, dst_hbm, ocnt_hbm, icnt_hbm, acc, zbuf, ones_b, idx_all,
          sem):
        c = lax.axis_index("c")
        s = lax.axis_index("s")

        @pl.when(c == 0)
        def _():
            pltpu.sync_copy(src_hbm.at[pl.ds(s * EPT_A, EPT_A)], idx_all)

        @pl.when(c == 1)
        def _():
            pltpu.sync_copy(dst_hbm.at[pl.ds(s * EPT_A, EPT_A)], idx_all)

        _zero_rows(zbuf, ROWS_PER_TILE, L)
        pltpu.sync_copy(zbuf, acc.at[pl.ds(s * ROWS_PER_TILE, ROWS_PER_TILE)])
        ones16 = jnp.ones((L,), jnp.float32)
        for i in range(K):
            ones_b[i, :] = ones16
        plsc.subcore_barrier()

        def gbody(g, _):
            for u in range(5):
                pltpu.async_copy(
                    ones_b, acc.at[idx_all.at[pl.ds((g * 5 + u) * K, K)]],
                    sem, add=True)
            for u in range(5):
                pltpu.make_async_copy(ones_b, acc.at[idx_all.at[pl.ds(0, K)]],
                                      sem).wait()
            return 0

        lax.fori_loop(0, NCH_A // 5, gbody, 0)
        plsc.subcore_barrier()
        row0 = s * ROWS_PER_TILE

        @pl.when(c == 0)
        def _():
            pltpu.sync_copy(acc.at[pl.ds(row0, ROWS_PER_TILE)],
                            ocnt_hbm.at[pl.ds(row0, ROWS_PER_TILE),
                                        pl.ds(0, L)])

        @pl.when(c == 1)
        def _():
            pltpu.sync_copy(acc.at[pl.ds(row0, ROWS_PER_TILE)],
                            icnt_hbm.at[pl.ds(row0, ROWS_PER_TILE),
                                        pl.ds(0, L)])

    return k(src1, dst1)


def _gidx_body(ei_ref, ord_ref, gidx_ref, src_ref, dst_ref):
    s = ei_ref[0]
    gidx_ref[...] = ord_ref[...] * N + s
    src_ref[...] = s
    dst_ref[...] = ei_ref[1]


def _gidx(ei3, ord2):
    R = E // D
    return pl.pallas_call(
        _gidx_body,
        out_shape=(
            jax.ShapeDtypeStruct((R, D), jnp.int32),
            jax.ShapeDtypeStruct((R, D), jnp.int32),
            jax.ShapeDtypeStruct((R, D), jnp.int32),
        ),
    )(ei3, ord2)


def _transform_body(feat_ref, cnt_ref, wt_ref, out_ref):
    cnt = cnt_ref[:, 0:1]
    scale = 1.0 / jnp.sqrt(jnp.maximum(cnt, 1.0))
    res = jnp.dot(feat_ref[...] * scale, wt_ref[...],
                  preferred_element_type=jnp.float32)
    for o in range(NORD):
        out_ref[o] = res[:, o * D:(o + 1) * D]


def _transform(feat, ocnt, wt):
    blk = 1000
    return pl.pallas_call(
        _transform_body,
        grid=(N // blk,),
        in_specs=[
            pl.BlockSpec((blk, D), lambda i: (i, 0)),
            pl.BlockSpec((blk, D), lambda i: (i, 0)),
            pl.BlockSpec((D, NORD * D), lambda i: (0, 0)),
        ],
        out_specs=pl.BlockSpec((NORD, blk, D), lambda i: (0, i, 0)),
        out_shape=jax.ShapeDtypeStruct((NORD, N, D), jnp.float32),
    )(feat, ocnt, wt)


def _gather_scatter(tt, gidx1, dst1):
    mesh = plsc.VectorSubcoreMesh(core_axis_name="c", subcore_axis_name="s")
    SEC, CPS = 5, NCH_C // 5
    EPS = CPS * K

    @functools.partial(
        pl.kernel,
        mesh=mesh,
        out_type=jax.ShapeDtypeStruct((NC, NP, D), jnp.float32),
        scratch_types=[
            pltpu.VMEM_SHARED((NP, D), jnp.float32),
            pltpu.VMEM((EPS,), jnp.int32),
            pltpu.VMEM((EPS,), jnp.int32),
            pltpu.VMEM((K, D), jnp.float32),
            pltpu.VMEM((K, D), jnp.float32),
            pltpu.VMEM((K, D), jnp.float32),
            pltpu.VMEM((K, D), jnp.float32),
            pltpu.SemaphoreType.DMA,
            pltpu.SemaphoreType.DMA,
            pltpu.SemaphoreType.DMA,
            pltpu.SemaphoreType.DMA,
        ],
        compiler_params=pltpu.CompilerParams(use_tc_tiling_on_sc=False),
    )
    def k(tt_hbm, gidx_hbm, dst_hbm, part_hbm,
          acc, gidx_sec, dst_sec, rows0, rows1, rows2, rows3,
          gs0, gs1, gs2, gs3):
        c = lax.axis_index("c")
        s = lax.axis_index("s")
        wid = s * NC + c
        row0 = s * ROWS_PER_TILE
        ebase = wid * EPW_C

        _zero_rows(rows0, K, D)
        for r in range(8):
            pltpu.sync_copy(rows0, acc.at[pl.ds(row0 + r * K, K)])
        plsc.subcore_barrier()

        def sbody(sec, _):
            pltpu.sync_copy(gidx_hbm.at[pl.ds(ebase + sec * EPS, EPS)],
                            gidx_sec)
            pltpu.sync_copy(dst_hbm.at[pl.ds(ebase + sec * EPS, EPS)],
                            dst_sec)
            pltpu.async_copy(tt_hbm.at[gidx_sec.at[pl.ds(0, K)]], rows0, gs0)
            pltpu.async_copy(tt_hbm.at[gidx_sec.at[pl.ds(K, K)]], rows1, gs1)
            pltpu.async_copy(tt_hbm.at[gidx_sec.at[pl.ds(2 * K, K)]], rows2,
                             gs2)

            def body(i, _):
                def step(rcur, gcur, rfar, gfar):
                    pltpu.make_async_copy(
                        tt_hbm.at[gidx_sec.at[pl.ds(i * K, K)]], rcur,
                        gcur).wait()

                    @pl.when(i + 3 < CPS)
                    def _():
                        pltpu.async_copy(
                            tt_hbm.at[gidx_sec.at[pl.ds((i + 3) * K, K)]],
                            rfar, gfar)

                    pltpu.sync_copy(rcur, acc.at[dst_sec.at[pl.ds(i * K, K)]],
                                    add=True)

                @pl.when(i % 4 == 0)
                def _():
                    step(rows0, gs0, rows3, gs3)

                @pl.when(i % 4 == 1)
                def _():
                    step(rows1, gs1, rows0, gs0)

                @pl.when(i % 4 == 2)
                def _():
                    step(rows2, gs2, rows1, gs1)

                @pl.when(i % 4 == 3)
                def _():
                    step(rows3, gs3, rows2, gs2)

                return 0

            lax.fori_loop(0, CPS, body, 0)
            return 0

        lax.fori_loop(0, SEC, sbody, 0)
        plsc.subcore_barrier()

        @pl.when(c == 0)
        def _():
            pltpu.sync_copy(acc.at[pl.ds(row0, ROWS_PER_TILE)],
                            part_hbm.at[0, pl.ds(row0, ROWS_PER_TILE)])

        @pl.when(c == 1)
        def _():
            pltpu.sync_copy(acc.at[pl.ds(row0, ROWS_PER_TILE)],
                            part_hbm.at[1, pl.ds(row0, ROWS_PER_TILE)])

    return k(tt, gidx1, dst1)


def _final_body(part_ref, cnt_ref, bias_ref, out_ref):
    p = part_ref[0] + part_ref[1]
    cnt = cnt_ref[:, 0:1]
    scale = 1.0 / jnp.sqrt(jnp.maximum(cnt, 1.0))
    out_ref[...] = p * scale + bias_ref[...]


def _finalize(part, icnt, bias2d):
    blk = 1000
    return pl.pallas_call(
        _final_body,
        grid=(N // blk,),
        in_specs=[
            pl.BlockSpec((NC, blk, D), lambda i: (0, i, 0)),
            pl.BlockSpec((blk, D), lambda i: (i, 0)),
            pl.BlockSpec((1, D), lambda i: (0, 0)),
        ],
        out_specs=pl.BlockSpec((blk, D), lambda i: (i, 0)),
        out_shape=jax.ShapeDtypeStruct((N, D), jnp.float32),
    )(part, icnt, bias2d)


def kernel(feat, edge_index, edge_order, o_emb_weight, bias):
    ei3 = edge_index.astype(jnp.int32).reshape(2, E // D, D)
    order = edge_order.astype(jnp.int32)
    gidx2, src2, dst2 = _gidx(ei3, order.reshape(E // D, D))
    src1 = src2.reshape(E)
    dst1 = dst2.reshape(E)
    ocnt, icnt = _degrees(src1, dst1)
    wt = o_emb_weight.reshape(NORD, D, D).transpose(2, 0, 1).reshape(D, NORD * D)
    t = _transform(feat, ocnt, wt)
    part = _gather_scatter(t.reshape(NORD * N, D), gidx2.reshape(E), dst1)
    return _finalize(part, icnt, bias.reshape(1, D))
